# baseline (speedup 1.0000x reference)
import functools

import jax
import jax.numpy as jnp
from jax import lax
from jax.experimental import pallas as pl
from jax.experimental.pallas import tpu as pltpu
from jax.experimental.pallas import tpu_sc as plsc

NC = 2
NS = 16
NW = NC * NS
CHUNK = 128


def _gather_call(B, D2):
  b_per_w = B // NW
  n_chunks = b_per_w // CHUNK
  mesh = plsc.VectorSubcoreMesh(core_axis_name="c", subcore_axis_name="s")

  @functools.partial(
      pl.kernel,
      mesh=mesh,
      out_type=jax.ShapeDtypeStruct((B, D2), jnp.float32),
      scratch_types=[
          pltpu.VMEM((n_chunks, CHUNK), jnp.int32),
          pltpu.VMEM((b_per_w, D2), jnp.float32),
          pltpu.SemaphoreType.DMA,
      ],
  )
  def k(ind_hbm, z_hbm, out_hbm, idx_v, rows_v, sem):
    wid = lax.axis_index("s") * NC + lax.axis_index("c")
    base = wid * b_per_w
    pltpu.sync_copy(ind_hbm.at[wid], idx_v)
    copies = []
    for j in range(n_chunks):
      copies.append(
          pltpu.async_copy(
              z_hbm.at[idx_v.at[j]],
              rows_v.at[pl.ds(j * CHUNK, CHUNK)],
              sem,
          )
      )
    for c in copies:
      c.wait()
    pltpu.sync_copy(rows_v, out_hbm.at[pl.ds(base, b_per_w)])

  return k


def kernel(ind, z):
  B, = ind.shape
  V, D = z.shape
  z2 = z.reshape(V // 2, 2 * D)
  ind2 = (ind // 2).reshape(NW, B // (NW * CHUNK), CHUNK)
  pairs = _gather_call(B, 2 * D)(ind2, z2)
  return jnp.where((ind % 2 == 1)[:, None], pairs[:, D:], pairs[:, :D])


# speedup vs baseline: 2.5209x; 2.5209x over previous
import functools

import jax
import jax.numpy as jnp
from jax import lax
from jax.experimental import pallas as pl
from jax.experimental.pallas import tpu as pltpu
from jax.experimental.pallas import tpu_sc as plsc

NC = 2
NS = 16
NW = NC * NS
NBUF = 4


def _gather_call(B, D, V):
  b_per_w = B // NW
  tail_start = (V // 128) * 128
  tail_len = V - tail_start
  last_block = tail_start // 128 - 1
  mesh = plsc.VectorSubcoreMesh(core_axis_name="c", subcore_axis_name="s")

  @functools.partial(
      pl.kernel,
      mesh=mesh,
      out_type=jax.ShapeDtypeStruct((B * 128,), jnp.float32),
      compiler_params=pltpu.CompilerParams(needs_layout_passes=False),
      scratch_types=[
          pltpu.SMEM((b_per_w,), jnp.int32),
          pltpu.VMEM((b_per_w,), jnp.int32),
          pltpu.VMEM((b_per_w * 128,), jnp.float32),
          pltpu.VMEM((D, tail_len), jnp.float32),
          *[pltpu.VMEM((D, 128), jnp.float32) for _ in range(NBUF)],
          *[pltpu.SemaphoreType.DMA for _ in range(NBUF)],
      ],
  )
  def k(ind_hbm, zt_hbm, outf_hbm, idx_s, idx_v, stag_v, tail_v, *bufs_sems):
    bufs = bufs_sems[:NBUF]
    sems = bufs_sems[NBUF:]
    wid = lax.axis_index("s") * NC + lax.axis_index("c")
    base = wid * b_per_w
    pltpu.sync_copy(ind_hbm.at[pl.ds(base, b_per_w)], idx_v)

    lanes = lax.iota(jnp.int32, 16)

    def scalarize(g, _):
      v = idx_v[pl.ds(pl.multiple_of(g * 16, 16), 16)]
      for u in range(16):
        idx_s[g * 16 + u] = jnp.sum(jnp.where(lanes == u, v, 0))
      return ()

    lax.fori_loop(0, b_per_w // 16, scalarize, ())
    if tail_len:
      pltpu.sync_copy(zt_hbm.at[:, pl.ds(tail_start, tail_len)], tail_v)

    rows = [lax.iota(jnp.int32, 16) + 16 * s for s in range(D // 16)]

    def fire(j, slot):
      q = idx_s[j]
      t = jnp.minimum(lax.shift_right_logical(q, 7), last_block)
      toff = pl.multiple_of(t * 128, 128)
      return pltpu.async_copy(
          zt_hbm.at[:, pl.ds(toff, 128)], bufs[slot], sems[slot]
      )

    for j in range(NBUF):
      fire(j, j)

    def body(g, _):
      for u in range(NBUF):
        j = g * NBUF + u
        pltpu.make_async_copy(
            zt_hbm.at[:, pl.ds(0, 128)], bufs[u], sems[u]
        ).wait()
        q = idx_s[j]
        in_main = q < tail_start
        off = pl.multiple_of(j * 128, 128)

        @pl.when(in_main)
        def _():
          l = jnp.broadcast_to(lax.rem(q, 128), (16,))
          for s in range(D // 16):
            stag_v[pl.ds(off + 16 * s, 16)] = plsc.load_gather(
                bufs[u], [rows[s], l]
            )

        if tail_len:
          @pl.when(jnp.logical_not(in_main))
          def _():
            l2 = jnp.broadcast_to(q - tail_start, (16,))
            for s in range(D // 16):
              stag_v[pl.ds(off + 16 * s, 16)] = plsc.load_gather(
                  tail_v, [rows[s], l2]
              )

        @pl.when(j + NBUF < b_per_w)
        def _():
          fire(j + NBUF, u)

      return ()

    lax.fori_loop(0, b_per_w // NBUF, body, ())
    pltpu.sync_copy(
        stag_v, outf_hbm.at[pl.ds(base * 128, b_per_w * 128)]
    )

  return k


def kernel(ind, z):
  B, = ind.shape
  V, D = z.shape
  out_flat = _gather_call(B, D, V)(ind, z.T)
  return out_flat.reshape(B, 128)[:, :D]
